# Initial kernel scaffold; baseline (speedup 1.0000x reference)
import jax
import jax.numpy as jnp
from jax.experimental import pallas as pl


def kernel(x):
    raise NotImplementedError("write your pallas kernel here")


# speedup vs baseline: 1.0179x; 1.0179x over previous
import jax
import jax.numpy as jnp
from jax.experimental import pallas as pl

_INDEX = 1024


def _copy_kernel(x_ref, o_ref):
    o_ref[...] = x_ref[:, _INDEX % 8, :]


def kernel(x):
    b, s, d = x.shape
    return pl.pallas_call(
        _copy_kernel,
        grid=(1,),
        in_specs=[pl.BlockSpec((b, 8, d), lambda i: (0, _INDEX // 8, 0))],
        out_specs=pl.BlockSpec((b, d), lambda i: (0, 0)),
        out_shape=jax.ShapeDtypeStruct((b, d), x.dtype),
    )(x)
